# Initial kernel scaffold; baseline (speedup 1.0000x reference)
import jax
import jax.numpy as jnp
from jax.experimental import pallas as pl


def kernel(x, W1, b1, W2, b2, Wg, We, be, Wp, bp):
    raise NotImplementedError("write your pallas kernel here")


# speedup vs baseline: 1.5603x; 1.5603x over previous
import functools

import jax
import jax.numpy as jnp
from jax.experimental import pallas as pl
from jax.experimental.pallas import tpu as pltpu

BT = 256


def _moe_body(E, NC, x_ref, W1_ref, b1_ref, W2_ref, b2_ref, Wg_ref, We_ref,
              be_ref, Wp_ref, bp_ref, out_ref):
    EP = Wg_ref.shape[1]
    NCP = Wp_ref.shape[1]
    x = x_ref[...]
    h = jnp.maximum(jnp.dot(x, W1_ref[...], preferred_element_type=jnp.float32)
                    + b1_ref[...], 0.0)
    h = jnp.maximum(jnp.dot(h, W2_ref[...], preferred_element_type=jnp.float32)
                    + b2_ref[...], 0.0)
    logits = jnp.dot(h, Wg_ref[...], preferred_element_type=jnp.float32)
    col = jax.lax.broadcasted_iota(jnp.int32, (BT, EP), 1)
    logits = jnp.where(col < E, logits, -1e30)
    m = jnp.max(logits, axis=1, keepdims=True)
    ex = jnp.exp(logits - m)
    gates = ex / jnp.sum(ex, axis=1, keepdims=True)
    m1 = jnp.max(gates, axis=1, keepdims=True)
    i1 = jnp.min(jnp.where(gates == m1, col, EP), axis=1, keepdims=True)
    g2 = jnp.where(col == i1, -1.0, gates)
    m2 = jnp.max(g2, axis=1, keepdims=True)
    i2 = jnp.min(jnp.where(g2 == m2, col, EP), axis=1, keepdims=True)
    denom = m1 + m2 + 1e-9
    w1 = m1 / denom
    w2 = m2 / denom
    acc = jnp.zeros((BT, W1_ref.shape[1]), dtype=jnp.float32)
    for e in range(E):
        we = jnp.where(i1 == e, w1, 0.0) + jnp.where(i2 == e, w2, 0.0)
        eo = jnp.dot(h, We_ref[e], preferred_element_type=jnp.float32) \
            + be_ref[e:e + 1, :]
        acc = acc + we * eo
    lg = jnp.dot(acc, Wp_ref[...], preferred_element_type=jnp.float32) + bp_ref[...]
    colc = jax.lax.broadcasted_iota(jnp.int32, (BT, NCP), 1)
    lg = jnp.where(colc < NC, lg, -1e30)
    mm = jnp.max(lg, axis=1, keepdims=True)
    lse = jnp.log(jnp.sum(jnp.exp(lg - mm), axis=1, keepdims=True)) + mm
    out_ref[...] = lg - lse


def kernel(x, W1, b1, W2, b2, Wg, We, be, Wp, bp):
    N = x.shape[0]
    D = W1.shape[0]
    E = Wg.shape[1]
    NC = Wp.shape[1]
    EP = 128
    NCP = ((NC + 127) // 128) * 128
    xf = x.reshape(N, D)
    Wg_p = jnp.pad(Wg, ((0, 0), (0, EP - E)))
    Wp_p = jnp.pad(Wp, ((0, 0), (0, NCP - NC)))
    bp_p = jnp.pad(bp, (0, NCP - NC)).reshape(1, NCP)
    b1r = b1.reshape(1, D)
    b2r = b2.reshape(1, D)

    grid = (N // BT,)
    full = lambda *s: pl.BlockSpec(s, lambda i: (0,) * len(s))
    out = pl.pallas_call(
        functools.partial(_moe_body, E, NC),
        grid=grid,
        in_specs=[
            pl.BlockSpec((BT, D), lambda i: (i, 0)),
            full(D, D),
            full(1, D),
            full(D, D),
            full(1, D),
            full(D, EP),
            full(E, D, D),
            full(E, D),
            full(D, NCP),
            full(1, NCP),
        ],
        out_specs=pl.BlockSpec((BT, NCP), lambda i: (i, 0)),
        out_shape=jax.ShapeDtypeStruct((N, NCP), jnp.float32),
        compiler_params=pltpu.CompilerParams(
            dimension_semantics=("arbitrary",),
        ),
    )(xf, W1, b1r, W2, b2r, Wg_p, We, be, Wp_p, bp_p)
    return out[:, :NC]


# speedup vs baseline: 1.5627x; 1.0015x over previous
import functools

import jax
import jax.numpy as jnp
from jax.experimental import pallas as pl
from jax.experimental.pallas import tpu as pltpu

BT = 256


def _moe_body(E, NC, x_ref, W1_ref, b1_ref, W2_ref, b2_ref, Wg_ref, We_ref,
              be_ref, Wp_ref, bp_ref, out_ref):
    EP = Wg_ref.shape[1]
    NCP = Wp_ref.shape[1]
    bf = jnp.bfloat16
    x = x_ref[...]
    h = jnp.maximum(jnp.dot(x.astype(bf), W1_ref[...].astype(bf),
                            preferred_element_type=jnp.float32)
                    + b1_ref[...], 0.0)
    h = jnp.maximum(jnp.dot(h.astype(bf), W2_ref[...].astype(bf),
                            preferred_element_type=jnp.float32)
                    + b2_ref[...], 0.0)
    logits = jnp.dot(h, Wg_ref[...], preferred_element_type=jnp.float32)
    col = jax.lax.broadcasted_iota(jnp.int32, (BT, EP), 1)
    logits = jnp.where(col < E, logits, -1e30)
    m = jnp.max(logits, axis=1, keepdims=True)
    ex = jnp.exp(logits - m)
    gates = ex / jnp.sum(ex, axis=1, keepdims=True)
    m1 = jnp.max(gates, axis=1, keepdims=True)
    i1 = jnp.min(jnp.where(gates == m1, col, EP), axis=1, keepdims=True)
    g2 = jnp.where(col == i1, -1.0, gates)
    m2 = jnp.max(g2, axis=1, keepdims=True)
    i2 = jnp.min(jnp.where(g2 == m2, col, EP), axis=1, keepdims=True)
    denom = m1 + m2 + 1e-9
    w1 = m1 / denom
    w2 = m2 / denom
    acc = jnp.zeros((BT, W1_ref.shape[1]), dtype=jnp.float32)
    for e in range(E):
        we = jnp.where(i1 == e, w1, 0.0) + jnp.where(i2 == e, w2, 0.0)
        eo = jnp.dot(h.astype(bf), We_ref[e].astype(bf),
                     preferred_element_type=jnp.float32) + be_ref[e:e + 1, :]
        acc = acc + we * eo
    lg = jnp.dot(acc.astype(bf), Wp_ref[...].astype(bf),
                 preferred_element_type=jnp.float32) + bp_ref[...]
    colc = jax.lax.broadcasted_iota(jnp.int32, (BT, NCP), 1)
    lg = jnp.where(colc < NC, lg, -1e30)
    mm = jnp.max(lg, axis=1, keepdims=True)
    lse = jnp.log(jnp.sum(jnp.exp(lg - mm), axis=1, keepdims=True)) + mm
    out_ref[...] = lg - lse


def kernel(x, W1, b1, W2, b2, Wg, We, be, Wp, bp):
    N = x.shape[0]
    D = W1.shape[0]
    E = Wg.shape[1]
    NC = Wp.shape[1]
    EP = 128
    NCP = ((NC + 127) // 128) * 128
    xf = x.reshape(N, D)
    Wg_p = jnp.pad(Wg, ((0, 0), (0, EP - E)))
    Wp_p = jnp.pad(Wp, ((0, 0), (0, NCP - NC)))
    bp_p = jnp.pad(bp, (0, NCP - NC)).reshape(1, NCP)
    b1r = b1.reshape(1, D)
    b2r = b2.reshape(1, D)

    grid = (N // BT,)
    full = lambda *s: pl.BlockSpec(s, lambda i: (0,) * len(s))
    out = pl.pallas_call(
        functools.partial(_moe_body, E, NC),
        grid=grid,
        in_specs=[
            pl.BlockSpec((BT, D), lambda i: (i, 0)),
            full(D, D),
            full(1, D),
            full(D, D),
            full(1, D),
            full(D, EP),
            full(E, D, D),
            full(E, D),
            full(D, NCP),
            full(1, NCP),
        ],
        out_specs=pl.BlockSpec((BT, NCP), lambda i: (i, 0)),
        out_shape=jax.ShapeDtypeStruct((N, NCP), jnp.float32),
        compiler_params=pltpu.CompilerParams(
            dimension_semantics=("arbitrary",),
        ),
    )(xf, W1, b1r, W2, b2r, Wg_p, We, be, Wp_p, bp_p)
    return out[:, :NC]
